# baseline (speedup 1.0000x reference)
import functools

import jax
import jax.numpy as jnp
from jax import lax
from jax.experimental import pallas as pl
from jax.experimental.pallas import tpu as pltpu
from jax.experimental.pallas import tpu_sc as plsc

B = 16
N = 1024
ROWS = B * N
D = 64
K = 1024
BLK = 1024
HALF_B = B // 2
HROWS = HALF_B * N
NBLK = HROWS // BLK

NC, NS = 2, 16
NW = NC * NS
BPW = HROWS // NW
WPB = N // BPW


def _make_tc_body(half):
    def _tc_distance_argmin(z_ref, cb2_ref, cn_ref, idx_ref, loss_ref):
        i = pl.program_id(0)
        z = z_ref[0]
        cb2 = cb2_ref[...] * -2.0
        s = lax.dot_general(z, cb2, (((1,), (1,)), ((), ())),
                            preferred_element_type=jnp.float32)
        zn = jnp.sum(z ** 2, axis=1, keepdims=True)
        d = zn + s + cn_ref[...]
        m = jnp.min(d, axis=1, keepdims=True)
        colf = lax.broadcasted_iota(jnp.int32, d.shape, 1).astype(jnp.float32)
        idxf = jnp.min(jnp.where(d <= m, colf, 65536.0), axis=1,
                       keepdims=True)
        idx_ref[0, 0, :] = jnp.transpose(idxf, (1, 0))[0].astype(jnp.int32)

        part = jnp.sum(m, axis=0, keepdims=True)

        @pl.when(i == 0)
        def _init():
            loss_ref[...] = part

        @pl.when(i > 0)
        def _acc():
            loss_ref[...] = loss_ref[...] + part

    del half
    return _tc_distance_argmin


def _tc_half(z, cb2, cn, half):
    base = half * HALF_B
    return pl.pallas_call(
        _make_tc_body(half),
        grid=(NBLK,),
        in_specs=[
            pl.BlockSpec((1, BLK, D),
                         lambda i: (base + i // (N // BLK), i % (N // BLK), 0)),
            pl.BlockSpec((K, D), lambda i: (0, 0)),
            pl.BlockSpec((1, K), lambda i: (0, 0)),
        ],
        out_specs=[
            pl.BlockSpec((1, 1, BLK), lambda i: (i, 0, 0)),
            pl.BlockSpec((1, 1), lambda i: (0, 0)),
        ],
        out_shape=[
            jax.ShapeDtypeStruct((NBLK, 1, BLK), jnp.int32),
            jax.ShapeDtypeStruct((1, 1), jnp.float32),
        ],
    )(z, cb2, cn)


def _sc_gather_body(cb_hbm, idx_hbm, out_hbm, idx_v, rows_v, sem):
    wid = lax.axis_index("s") * NC + lax.axis_index("c")
    base = wid * BPW
    pltpu.sync_copy(idx_hbm.at[pl.ds(base, BPW)], idx_v)
    pltpu.async_copy(cb_hbm.at[idx_v], rows_v, sem).wait()
    pltpu.sync_copy(rows_v, out_hbm.at[pl.ds(base, BPW)])


@functools.cache
def _make_sc_gather():
    return pl.kernel(
        _sc_gather_body,
        out_type=jax.ShapeDtypeStruct((HROWS, D), jnp.float32),
        mesh=plsc.VectorSubcoreMesh(core_axis_name="c", subcore_axis_name="s",
                                    num_cores=NC, num_subcores=NS),
        scratch_types=[
            pltpu.VMEM((BPW,), jnp.int32),
            pltpu.VMEM((BPW, D), jnp.float32),
            pltpu.SemaphoreType.DMA,
        ],
        compiler_params=pltpu.CompilerParams(use_tc_tiling_on_sc=False),
    )


@jax.jit
def kernel(z, codebook):
    cn = jnp.sum(codebook ** 2, axis=1)[None, :]
    gather = _make_sc_gather()

    idx0_3, p0 = _tc_half(z, codebook, cn, 0)
    idx0 = idx0_3.reshape(HROWS)
    zq0 = gather(codebook, idx0)
    idx1_3, p1 = _tc_half(z, codebook, cn, 1)
    idx1 = idx1_3.reshape(HROWS)
    zq1 = gather(codebook, idx1)

    zq = jnp.concatenate([zq0, zq1], axis=0).reshape(B, N, D)
    idx = jnp.concatenate([idx0, idx1]).reshape(B, N)
    loss = (p0[0, 0] + p1[0, 0]) * (1.25 / (ROWS * D))
    return zq, loss, idx


# speedup vs baseline: 1.0139x; 1.0139x over previous
import functools

import jax
import jax.numpy as jnp
from jax import lax
from jax.experimental import pallas as pl
from jax.experimental.pallas import tpu as pltpu
from jax.experimental.pallas import tpu_sc as plsc

B = 16
N = 1024
ROWS = B * N
D = 64
K = 1024
BLK = 1024
HALF_B = B // 2
HROWS = HALF_B * N
NBLK = HROWS // BLK

NC, NS = 2, 16
NW = NC * NS
BPW = HROWS // NW
WPB = N // BPW


def _make_tc_body(half):
    def _tc_distance_argmin(z_ref, cb2_ref, cn_ref, idx_ref, loss_ref):
        i = pl.program_id(0)
        z = z_ref[0]
        cb2 = cb2_ref[...]
        s = lax.dot_general(z, cb2, (((1,), (1,)), ((), ())),
                            preferred_element_type=jnp.float32)
        zn = jnp.sum(z ** 2, axis=1, keepdims=True)
        d = zn + s + cn_ref[...]
        m = jnp.min(d, axis=1, keepdims=True)
        colf = lax.broadcasted_iota(jnp.int32, d.shape, 1).astype(jnp.float32)
        idxf = jnp.min(jnp.where(d <= m, colf, 65536.0), axis=1,
                       keepdims=True)
        idx_ref[0, 0, :] = jnp.transpose(idxf, (1, 0))[0].astype(jnp.int32)

        part = jnp.sum(m, axis=0, keepdims=True)

        @pl.when(i == 0)
        def _init():
            loss_ref[...] = part

        @pl.when(i > 0)
        def _acc():
            loss_ref[...] = loss_ref[...] + part

    del half
    return _tc_distance_argmin


def _tc_half(z, cb2, cn, half):
    base = half * HALF_B
    return pl.pallas_call(
        _make_tc_body(half),
        grid=(NBLK,),
        in_specs=[
            pl.BlockSpec((1, BLK, D),
                         lambda i: (base + i // (N // BLK), i % (N // BLK), 0)),
            pl.BlockSpec((K, D), lambda i: (0, 0)),
            pl.BlockSpec((1, K), lambda i: (0, 0)),
        ],
        out_specs=[
            pl.BlockSpec((1, 1, BLK), lambda i: (i, 0, 0)),
            pl.BlockSpec((1, 1), lambda i: (0, 0)),
        ],
        out_shape=[
            jax.ShapeDtypeStruct((NBLK, 1, BLK), jnp.int32),
            jax.ShapeDtypeStruct((1, 1), jnp.float32),
        ],
    )(z, cb2, cn)


def _sc_gather_body(cb_hbm, idx_hbm, out_hbm, idx_v, rows_v, sem):
    wid = lax.axis_index("s") * NC + lax.axis_index("c")
    base = wid * BPW
    pltpu.sync_copy(idx_hbm.at[pl.ds(base, BPW)], idx_v)
    pltpu.async_copy(cb_hbm.at[idx_v], rows_v, sem).wait()
    pltpu.sync_copy(rows_v, out_hbm.at[pl.ds(base, BPW)])


@functools.cache
def _make_sc_gather():
    return pl.kernel(
        _sc_gather_body,
        out_type=jax.ShapeDtypeStruct((HROWS, D), jnp.float32),
        mesh=plsc.VectorSubcoreMesh(core_axis_name="c", subcore_axis_name="s",
                                    num_cores=NC, num_subcores=NS),
        scratch_types=[
            pltpu.VMEM((BPW,), jnp.int32),
            pltpu.VMEM((BPW, D), jnp.float32),
            pltpu.SemaphoreType.DMA,
        ],
        compiler_params=pltpu.CompilerParams(use_tc_tiling_on_sc=False),
    )


@jax.jit
def kernel(z, codebook):
    cn = jnp.sum(codebook ** 2, axis=1)[None, :]
    cb2 = -2.0 * codebook
    gather = _make_sc_gather()

    idx0_3, p0 = _tc_half(z, cb2, cn, 0)
    idx0 = idx0_3.reshape(HROWS)
    zq0 = gather(codebook, idx0)
    idx1_3, p1 = _tc_half(z, cb2, cn, 1)
    idx1 = idx1_3.reshape(HROWS)
    zq1 = gather(codebook, idx1)

    zq = jnp.concatenate([zq0, zq1], axis=0).reshape(B, N, D)
    idx = jnp.concatenate([idx0, idx1]).reshape(B, N)
    loss = (p0[0, 0] + p1[0, 0]) * (1.25 / (ROWS * D))
    return zq, loss, idx
